# Initial kernel scaffold; baseline (speedup 1.0000x reference)
import jax
import jax.numpy as jnp
from jax.experimental import pallas as pl


def kernel(history_data, emb, layers, final_norm, od_w, od_b):
    raise NotImplementedError("write your pallas kernel here")


# speedup vs baseline: 1.3330x; 1.3330x over previous
import math
import functools

import jax
import jax.numpy as jnp
from jax.experimental import pallas as pl
from jax.experimental.pallas import tpu as pltpu

B = 2
S = 307
SP = 320
T = B * SP
D = 2048
H = 16
HD = 128
E = 8
NEG = -1e9
BF = jnp.bfloat16


def _rms(x, g):
    v = jnp.mean(x * x, axis=1, keepdims=True)
    return x * jax.lax.rsqrt(v + 1e-6) * g


def _embed_body(hist_ref, cw_ref, cb_ref, lg_ref, lb_ref, lw_ref, lb2_ref, o_ref):
    s = hist_ref[...]
    a = jnp.maximum(s * cw_ref[...] + cb_ref[...], 0.0)
    mu = jnp.mean(a, axis=1, keepdims=True)
    var = jnp.mean((a - mu) * (a - mu), axis=1, keepdims=True)
    nrm = (a - mu) * jax.lax.rsqrt(var + 1e-5) * lg_ref[...] + lb_ref[...]
    n_last = nrm[:, -1:]
    o_ref[...] = n_last * lw_ref[...] + lb2_ref[...]


def _rms_mm_body(x_ref, g_ref, w_ref, o_ref):
    xn = _rms(x_ref[...], g_ref[...]).astype(BF)
    w = w_ref[...].astype(BF)
    o_ref[...] = jax.lax.dot_general(
        xn, w, (((1,), (1,)), ((), ())), preferred_element_type=jnp.float32)


def _mm_add_body(x_ref, w_ref, r_ref, o_ref):
    x = x_ref[...].astype(BF)
    w = w_ref[...].astype(BF)
    y = jax.lax.dot_general(x, w, (((1,), (1,)), ((), ())),
                            preferred_element_type=jnp.float32)
    o_ref[...] = y + r_ref[...]


def _rope(x):
    pos = jax.lax.broadcasted_iota(jnp.int32, (SP, HD // 2), 0).astype(jnp.float32)
    idx = jax.lax.broadcasted_iota(jnp.int32, (SP, HD // 2), 1).astype(jnp.float32)
    inv = jnp.exp(idx * (-2.0 * math.log(10000.0) / HD))
    ang = pos * inv
    c = jnp.cos(ang)
    sn = jnp.sin(ang)
    cos = jnp.concatenate([c, c], axis=1)
    sin = jnp.concatenate([sn, sn], axis=1)
    x1 = x[:, :HD // 2]
    x2 = x[:, HD // 2:]
    return x * cos + jnp.concatenate([-x2, x1], axis=1) * sin


def _attn_body(q_ref, k_ref, v_ref, o_ref):
    q = _rope(q_ref[0])
    k = _rope(k_ref[0])
    s = jax.lax.dot_general(q, k, (((1,), (1,)), ((), ())),
                            preferred_element_type=jnp.float32)
    s = s * (1.0 / math.sqrt(HD))
    row = jax.lax.broadcasted_iota(jnp.int32, (SP, SP), 0)
    col = jax.lax.broadcasted_iota(jnp.int32, (SP, SP), 1)
    s = s + jnp.where(col > row, NEG, 0.0)
    m = jnp.max(s, axis=1, keepdims=True)
    p = jnp.exp(s - m)
    a = p / jnp.sum(p, axis=1, keepdims=True)
    o_ref[0] = jax.lax.dot_general(a, v_ref[0], (((1,), (0,)), ((), ())),
                                   preferred_element_type=jnp.float32)


def _ffn_body(x_ref, g_ref, wg_ref, wu_ref, wd_ref, r_ref, o_ref):
    j = pl.program_id(0)
    xn = _rms(x_ref[...], g_ref[...]).astype(BF)
    g = jax.lax.dot_general(xn, wg_ref[...].astype(BF), (((1,), (1,)), ((), ())),
                            preferred_element_type=jnp.float32)
    u = jax.lax.dot_general(xn, wu_ref[...].astype(BF), (((1,), (1,)), ((), ())),
                            preferred_element_type=jnp.float32)
    a = (g * jax.lax.logistic(g) * u).astype(BF)
    contrib = jax.lax.dot_general(a, wd_ref[...].astype(BF), (((1,), (1,)), ((), ())),
                                  preferred_element_type=jnp.float32)

    @pl.when(j == 0)
    def _():
        o_ref[...] = r_ref[...] + contrib

    @pl.when(j != 0)
    def _():
        o_ref[...] = o_ref[...] + contrib


def _moe_body(x_ref, g_ref, rw_ref, eg_ref, eu_ref, ed_ref, r_ref, o_ref, gates_ref):
    e = pl.program_id(0)
    xn = _rms(x_ref[...], g_ref[...])

    @pl.when(e == 0)
    def _():
        logits = jax.lax.dot_general(xn, rw_ref[...], (((1,), (1,)), ((), ())),
                                     preferred_element_type=jnp.float32)
        mx = jnp.max(logits, axis=1, keepdims=True)
        ex = jnp.exp(logits - mx)
        sm = ex / jnp.sum(ex, axis=1, keepdims=True)
        cols = jax.lax.broadcasted_iota(jnp.int32, (T, E), 1)
        i1 = jnp.argmax(sm, axis=1)
        oh1 = cols == i1[:, None]
        m1 = jnp.max(sm, axis=1, keepdims=True)
        sm2 = jnp.where(oh1, -jnp.inf, sm)
        i2 = jnp.argmax(sm2, axis=1)
        oh2 = cols == i2[:, None]
        m2 = jnp.max(sm2, axis=1, keepdims=True)
        gates_ref[...] = jnp.where(oh1, m1, 0.0) + jnp.where(oh2, m2, 0.0)

    gates = gates_ref[...]
    cols = jax.lax.broadcasted_iota(jnp.int32, (T, E), 1)
    we = jnp.sum(jnp.where(cols == e, gates, 0.0), axis=1, keepdims=True)

    xb = xn.astype(BF)
    g = jax.lax.dot_general(xb, eg_ref[0].astype(BF), (((1,), (1,)), ((), ())),
                            preferred_element_type=jnp.float32)
    u = jax.lax.dot_general(xb, eu_ref[0].astype(BF), (((1,), (1,)), ((), ())),
                            preferred_element_type=jnp.float32)
    a = (g * jax.lax.logistic(g) * u).astype(BF)
    ye = jax.lax.dot_general(a, ed_ref[0].astype(BF), (((1,), (1,)), ((), ())),
                             preferred_element_type=jnp.float32)
    contrib = we * ye

    @pl.when(e == 0)
    def _():
        o_ref[...] = r_ref[...] + contrib

    @pl.when(e != 0)
    def _():
        o_ref[...] = o_ref[...] + contrib


def _head_body(x_ref, g_ref, w_ref, b_ref, o_ref):
    xn = _rms(x_ref[...], g_ref[...]).astype(BF)
    y = jax.lax.dot_general(xn, w_ref[...].astype(BF), (((1,), (1,)), ((), ())),
                            preferred_element_type=jnp.float32)
    o_ref[...] = y + b_ref[...]


def _full(shape):
    return pl.BlockSpec(shape, lambda *a: (0,) * len(shape))


def _embed(hist_pad, emb):
    return pl.pallas_call(
        _embed_body,
        out_shape=jax.ShapeDtypeStruct((T, D), jnp.float32),
        in_specs=[_full((T, 1))] + [_full(s) for s in
                  [(1, 64), (1, 64), (1, 64), (1, 64), (1, D), (1, D)]],
        out_specs=_full((T, D)),
    )(hist_pad,
      emb['conv_w'].reshape(1, 64), emb['conv_b'].reshape(1, 64),
      emb['ln_g'].reshape(1, 64), emb['ln_b'].reshape(1, 64),
      emb['lin_w'].reshape(1, D), emb['lin_b'].reshape(1, D))


def _rms_mm(x, g, w, bn=512):
    n = w.shape[0]
    return pl.pallas_call(
        _rms_mm_body,
        grid=(n // bn,),
        out_shape=jax.ShapeDtypeStruct((T, n), jnp.float32),
        in_specs=[_full((T, D)), _full((1, D)),
                  pl.BlockSpec((bn, D), lambda j: (j, 0))],
        out_specs=pl.BlockSpec((T, bn), lambda j: (0, j)),
        compiler_params=pltpu.CompilerParams(dimension_semantics=("parallel",)),
    )(x, g.reshape(1, D), w)


def _mm_add(x, w, r, bn=512):
    return pl.pallas_call(
        _mm_add_body,
        grid=(D // bn,),
        out_shape=jax.ShapeDtypeStruct((T, D), jnp.float32),
        in_specs=[_full((T, D)),
                  pl.BlockSpec((bn, D), lambda j: (j, 0)),
                  pl.BlockSpec((T, bn), lambda j: (0, j))],
        out_specs=pl.BlockSpec((T, bn), lambda j: (0, j)),
        compiler_params=pltpu.CompilerParams(dimension_semantics=("parallel",)),
    )(x, w, r)


def _attention(qkv):
    spec_q = pl.BlockSpec((1, SP, HD), lambda b, h: (b, 0, h))
    spec_k = pl.BlockSpec((1, SP, HD), lambda b, h: (b, 0, H + h))
    spec_v = pl.BlockSpec((1, SP, HD), lambda b, h: (b, 0, 2 * H + h))
    return pl.pallas_call(
        _attn_body,
        grid=(B, H),
        out_shape=jax.ShapeDtypeStruct((B, SP, D), jnp.float32),
        in_specs=[spec_q, spec_k, spec_v],
        out_specs=pl.BlockSpec((1, SP, HD), lambda b, h: (b, 0, h)),
        compiler_params=pltpu.CompilerParams(
            dimension_semantics=("parallel", "parallel")),
    )(qkv, qkv, qkv)


def _ffn(x, g, wg, wu, wd, r, bf=512):
    ff = wg.shape[0]
    return pl.pallas_call(
        _ffn_body,
        grid=(ff // bf,),
        out_shape=jax.ShapeDtypeStruct((T, D), jnp.float32),
        in_specs=[_full((T, D)), _full((1, D)),
                  pl.BlockSpec((bf, D), lambda j: (j, 0)),
                  pl.BlockSpec((bf, D), lambda j: (j, 0)),
                  pl.BlockSpec((D, bf), lambda j: (0, j)),
                  _full((T, D))],
        out_specs=_full((T, D)),
        compiler_params=pltpu.CompilerParams(dimension_semantics=("arbitrary",)),
    )(x, g.reshape(1, D), wg, wu, wd, r)


def _moe(x, g, rw, eg, eu, ed, r):
    mf = eg.shape[1]
    return pl.pallas_call(
        _moe_body,
        grid=(E,),
        out_shape=jax.ShapeDtypeStruct((T, D), jnp.float32),
        in_specs=[_full((T, D)), _full((1, D)), _full((E, D)),
                  pl.BlockSpec((1, mf, D), lambda e: (e, 0, 0)),
                  pl.BlockSpec((1, mf, D), lambda e: (e, 0, 0)),
                  pl.BlockSpec((1, D, mf), lambda e: (e, 0, 0)),
                  _full((T, D))],
        out_specs=_full((T, D)),
        scratch_shapes=[pltpu.VMEM((T, E), jnp.float32)],
        compiler_params=pltpu.CompilerParams(dimension_semantics=("arbitrary",)),
    )(x, g.reshape(1, D), rw, eg, eu, ed, r)


def _head(x, g, w, b):
    n = w.shape[0]
    return pl.pallas_call(
        _head_body,
        out_shape=jax.ShapeDtypeStruct((T, n), jnp.float32),
        in_specs=[_full((T, D)), _full((1, D)), _full((n, D)), _full((1, n))],
        out_specs=_full((T, n)),
    )(x, g.reshape(1, D), w, b.reshape(1, n))


def kernel(history_data, emb, layers, final_norm, od_w, od_b):
    hist_pad = jnp.pad(history_data, ((0, 0), (0, SP - S))).reshape(T, 1)
    h = _embed(hist_pad, emb)

    for lp in layers:
        wqkv = jnp.concatenate([lp['q_w'], lp['k_w'], lp['v_w']], axis=0)
        qkv = _rms_mm(h, lp['attn_norm'], wqkv)
        attn = _attention(qkv.reshape(B, SP, 3 * D))
        h = _mm_add(attn.reshape(T, D), lp['o_w'], h)
        if 'router_w' in lp:
            h2 = _ffn(h, lp['ffn_norm'], lp['s_gate'], lp['s_up'], lp['s_down'], h)
            h = _moe(h, lp['ffn_norm'], lp['router_w'],
                     lp['e_gate'], lp['e_up'], lp['e_down'], h2)
        else:
            h = _ffn(h, lp['ffn_norm'], lp['gate_w'], lp['up_w'], lp['down_w'], h)

    y = _head(h, final_norm, od_w, od_b)
    return y.reshape(B, SP, S)[:, :S, :]


# speedup vs baseline: 1.4171x; 1.0631x over previous
import math

import jax
import jax.numpy as jnp
from jax.experimental import pallas as pl
from jax.experimental.pallas import tpu as pltpu

B = 2
S = 307
SP = 320
T = B * SP
D = 2048
H = 16
HD = 128
E = 8
NEG = -1e9


def _rms(x, g):
    v = jnp.mean(x * x, axis=1, keepdims=True)
    return x * jax.lax.rsqrt(v + 1e-6) * g


def _dot_t(a, b):
    return jax.lax.dot_general(a, b, (((1,), (1,)), ((), ())),
                               preferred_element_type=jnp.float32)


def _embed_body(hist_ref, cw_ref, cb_ref, lg_ref, lb_ref, lw_ref, lb2_ref, o_ref):
    s = hist_ref[...]
    a = jnp.maximum(s * cw_ref[...] + cb_ref[...], 0.0)
    mu = jnp.mean(a, axis=1, keepdims=True)
    var = jnp.mean((a - mu) * (a - mu), axis=1, keepdims=True)
    nrm = (a - mu) * jax.lax.rsqrt(var + 1e-5) * lg_ref[...] + lb_ref[...]
    n_last = nrm[:, -1:]
    o_ref[...] = n_last * lw_ref[...] + lb2_ref[...]


def _trig_body(ct_ref, st_ref):
    pos = jax.lax.broadcasted_iota(jnp.int32, (SP, HD), 0).astype(jnp.float32)
    col = jax.lax.broadcasted_iota(jnp.int32, (SP, HD), 1)
    fidx = jnp.where(col >= HD // 2, col - HD // 2, col).astype(jnp.float32)
    inv = jnp.exp(fidx * (-2.0 * math.log(10000.0) / HD))
    ang = pos * inv
    ct_ref[...] = jnp.cos(ang)
    st_ref[...] = jnp.sin(ang)


def _rms_mm_body(x_ref, g_ref, w_ref, o_ref, xn_ref):
    @pl.when(pl.program_id(0) == 0)
    def _():
        xn_ref[...] = _rms(x_ref[...], g_ref[...])

    o_ref[...] = _dot_t(xn_ref[...], w_ref[...])


def _rot(x):
    return jnp.concatenate([-x[:, HD // 2:], x[:, :HD // 2]], axis=1)


def _attn_body(q_ref, k_ref, v_ref, ct_ref, st_ref, ow_ref, r_ref, o_ref):
    h = pl.program_id(1)
    cos = ct_ref[...]
    sin = st_ref[...]
    q = q_ref[0] * (1.0 / math.sqrt(HD))
    qr = q * cos + _rot(q) * sin
    k = k_ref[0]
    kr = k * cos + _rot(k) * sin
    s = _dot_t(qr, kr)
    row = jax.lax.broadcasted_iota(jnp.int32, (SP, SP), 0)
    col = jax.lax.broadcasted_iota(jnp.int32, (SP, SP), 1)
    s = s + jnp.where(col > row, NEG, 0.0)
    m = jnp.max(s, axis=1, keepdims=True)
    p = jnp.exp(s - m)
    a = p * (1.0 / jnp.sum(p, axis=1, keepdims=True))
    o = jax.lax.dot_general(a, v_ref[0], (((1,), (0,)), ((), ())),
                            preferred_element_type=jnp.float32)
    contrib = _dot_t(o, ow_ref[...])

    @pl.when(h == 0)
    def _():
        o_ref[0] = r_ref[0] + contrib

    @pl.when(h != 0)
    def _():
        o_ref[0] = o_ref[0] + contrib


def _ffn_body(x_ref, g_ref, wg_ref, wu_ref, wd_ref, r_ref, o_ref, xn_ref):
    j = pl.program_id(0)

    @pl.when(j == 0)
    def _():
        xn_ref[...] = _rms(x_ref[...], g_ref[...])

    xn = xn_ref[...]
    g = _dot_t(xn, wg_ref[...])
    u = _dot_t(xn, wu_ref[...])
    a = g * jax.lax.logistic(g) * u
    contrib = _dot_t(a, wd_ref[...])

    @pl.when(j == 0)
    def _():
        o_ref[...] = r_ref[...] + contrib

    @pl.when(j != 0)
    def _():
        o_ref[...] = o_ref[...] + contrib


def _moe_body(x_ref, g_ref, rw_ref, eg_ref, eu_ref, ed_ref, r_ref, o_ref,
              xn_ref, gates_ref):
    e = pl.program_id(0)

    @pl.when(e == 0)
    def _():
        xn = _rms(x_ref[...], g_ref[...])
        xn_ref[...] = xn
        logits = _dot_t(xn, rw_ref[...])
        mx = jnp.max(logits, axis=1, keepdims=True)
        ex = jnp.exp(logits - mx)
        sm = ex * (1.0 / jnp.sum(ex, axis=1, keepdims=True))
        cols = jax.lax.broadcasted_iota(jnp.int32, (T, E), 1)
        i1 = jnp.argmax(sm, axis=1)
        oh1 = cols == i1[:, None]
        m1 = jnp.max(sm, axis=1, keepdims=True)
        sm2 = jnp.where(oh1, -jnp.inf, sm)
        i2 = jnp.argmax(sm2, axis=1)
        oh2 = cols == i2[:, None]
        m2 = jnp.max(sm2, axis=1, keepdims=True)
        gates_ref[...] = jnp.where(oh1, m1, 0.0) + jnp.where(oh2, m2, 0.0)

    xn = xn_ref[...]
    gates = gates_ref[...]
    cols = jax.lax.broadcasted_iota(jnp.int32, (T, E), 1)
    we = jnp.sum(jnp.where(cols == e, gates, 0.0), axis=1, keepdims=True)

    g = _dot_t(xn, eg_ref[0])
    u = _dot_t(xn, eu_ref[0])
    a = g * jax.lax.logistic(g) * u
    ye = _dot_t(a, ed_ref[0])
    contrib = we * ye

    @pl.when(e == 0)
    def _():
        o_ref[...] = r_ref[...] + contrib

    @pl.when(e != 0)
    def _():
        o_ref[...] = o_ref[...] + contrib


def _head_body(x_ref, g_ref, w_ref, b_ref, o_ref):
    xn = _rms(x_ref[...], g_ref[...])
    o_ref[...] = _dot_t(xn, w_ref[...]) + b_ref[...]


def _full(shape):
    return pl.BlockSpec(shape, lambda *a: (0,) * len(shape))


def _embed(hist_pad, emb):
    return pl.pallas_call(
        _embed_body,
        out_shape=jax.ShapeDtypeStruct((T, D), jnp.float32),
        in_specs=[_full((T, 1))] + [_full(s) for s in
                  [(1, 64), (1, 64), (1, 64), (1, 64), (1, D), (1, D)]],
        out_specs=_full((T, D)),
    )(hist_pad,
      emb['conv_w'].reshape(1, 64), emb['conv_b'].reshape(1, 64),
      emb['ln_g'].reshape(1, 64), emb['ln_b'].reshape(1, 64),
      emb['lin_w'].reshape(1, D), emb['lin_b'].reshape(1, D))


def _trig():
    return pl.pallas_call(
        _trig_body,
        out_shape=[jax.ShapeDtypeStruct((SP, HD), jnp.float32),
                   jax.ShapeDtypeStruct((SP, HD), jnp.float32)],
        out_specs=[_full((SP, HD)), _full((SP, HD))],
    )()


def _rms_mm(x, g, w, bn=512):
    n = w.shape[0]
    return pl.pallas_call(
        _rms_mm_body,
        grid=(n // bn,),
        out_shape=jax.ShapeDtypeStruct((T, n), jnp.float32),
        in_specs=[_full((T, D)), _full((1, D)),
                  pl.BlockSpec((bn, D), lambda j: (j, 0))],
        out_specs=pl.BlockSpec((T, bn), lambda j: (0, j)),
        scratch_shapes=[pltpu.VMEM((T, D), jnp.float32)],
        compiler_params=pltpu.CompilerParams(dimension_semantics=("arbitrary",)),
    )(x, g.reshape(1, D), w)


def _attention(qkv, ct, st, ow, r):
    spec_q = pl.BlockSpec((1, SP, HD), lambda b, h: (b, 0, h))
    spec_k = pl.BlockSpec((1, SP, HD), lambda b, h: (b, 0, H + h))
    spec_v = pl.BlockSpec((1, SP, HD), lambda b, h: (b, 0, 2 * H + h))
    return pl.pallas_call(
        _attn_body,
        grid=(B, H),
        out_shape=jax.ShapeDtypeStruct((B, SP, D), jnp.float32),
        in_specs=[spec_q, spec_k, spec_v,
                  _full((SP, HD)), _full((SP, HD)),
                  pl.BlockSpec((D, HD), lambda b, h: (0, h)),
                  pl.BlockSpec((1, SP, D), lambda b, h: (b, 0, 0))],
        out_specs=pl.BlockSpec((1, SP, D), lambda b, h: (b, 0, 0)),
        compiler_params=pltpu.CompilerParams(
            dimension_semantics=("arbitrary", "arbitrary")),
    )(qkv, qkv, qkv, ct, st, ow, r)


def _ffn(x, g, wg, wu, wd, r, bf=512):
    ff = wg.shape[0]
    return pl.pallas_call(
        _ffn_body,
        grid=(ff // bf,),
        out_shape=jax.ShapeDtypeStruct((T, D), jnp.float32),
        in_specs=[_full((T, D)), _full((1, D)),
                  pl.BlockSpec((bf, D), lambda j: (j, 0)),
                  pl.BlockSpec((bf, D), lambda j: (j, 0)),
                  pl.BlockSpec((D, bf), lambda j: (0, j)),
                  _full((T, D))],
        out_specs=_full((T, D)),
        scratch_shapes=[pltpu.VMEM((T, D), jnp.float32)],
        compiler_params=pltpu.CompilerParams(dimension_semantics=("arbitrary",)),
    )(x, g.reshape(1, D), wg, wu, wd, r)


def _moe(x, g, rw, eg, eu, ed, r):
    mf = eg.shape[1]
    return pl.pallas_call(
        _moe_body,
        grid=(E,),
        out_shape=jax.ShapeDtypeStruct((T, D), jnp.float32),
        in_specs=[_full((T, D)), _full((1, D)), _full((E, D)),
                  pl.BlockSpec((1, mf, D), lambda e: (e, 0, 0)),
                  pl.BlockSpec((1, mf, D), lambda e: (e, 0, 0)),
                  pl.BlockSpec((1, D, mf), lambda e: (e, 0, 0)),
                  _full((T, D))],
        out_specs=_full((T, D)),
        scratch_shapes=[pltpu.VMEM((T, D), jnp.float32),
                        pltpu.VMEM((T, E), jnp.float32)],
        compiler_params=pltpu.CompilerParams(dimension_semantics=("arbitrary",)),
    )(x, g.reshape(1, D), rw, eg, eu, ed, r)


def _head(x, g, w, b):
    n = w.shape[0]
    return pl.pallas_call(
        _head_body,
        out_shape=jax.ShapeDtypeStruct((T, n), jnp.float32),
        in_specs=[_full((T, D)), _full((1, D)), _full((n, D)), _full((1, n))],
        out_specs=_full((T, n)),
    )(x, g.reshape(1, D), w, b.reshape(1, n))


def kernel(history_data, emb, layers, final_norm, od_w, od_b):
    hist_pad = jnp.pad(history_data, ((0, 0), (0, SP - S))).reshape(T, 1)
    h = _embed(hist_pad, emb)
    ct, st = _trig()

    for lp in layers:
        wqkv = jnp.concatenate([lp['q_w'], lp['k_w'], lp['v_w']], axis=0)
        qkv = _rms_mm(h, lp['attn_norm'], wqkv)
        h = _attention(qkv.reshape(B, SP, 3 * D), ct, st,
                       lp['o_w'], h.reshape(B, SP, D)).reshape(T, D)
        if 'router_w' in lp:
            h2 = _ffn(h, lp['ffn_norm'], lp['s_gate'], lp['s_up'], lp['s_down'], h)
            h = _moe(h, lp['ffn_norm'], lp['router_w'],
                     lp['e_gate'], lp['e_up'], lp['e_down'], h2)
        else:
            h = _ffn(h, lp['ffn_norm'], lp['gate_w'], lp['up_w'], lp['down_w'], h)

    y = _head(h, final_norm, od_w, od_b)
    return y.reshape(B, SP, S)[:, :S, :]


# speedup vs baseline: 1.4766x; 1.0419x over previous
import math

import jax
import jax.numpy as jnp
from jax.experimental import pallas as pl
from jax.experimental.pallas import tpu as pltpu

B = 2
S = 307
SP = 320
T = B * SP
D = 2048
H = 16
HD = 128
E = 8
NS = 4
NEG = -1e9


def _rms(x, g):
    v = jnp.mean(x * x, axis=1, keepdims=True)
    return x * jax.lax.rsqrt(v + 1e-6) * g


def _dot_t(a, b):
    return jax.lax.dot_general(a, b, (((1,), (1,)), ((), ())),
                               preferred_element_type=jnp.float32)


def _embed_body(hist_ref, cw_ref, cb_ref, lg_ref, lb_ref, lw_ref, lb2_ref,
                o_ref, ct_ref, st_ref):
    s = hist_ref[...]
    a = jnp.maximum(s * cw_ref[...] + cb_ref[...], 0.0)
    mu = jnp.mean(a, axis=1, keepdims=True)
    var = jnp.mean((a - mu) * (a - mu), axis=1, keepdims=True)
    nrm = (a - mu) * jax.lax.rsqrt(var + 1e-5) * lg_ref[...] + lb_ref[...]
    n_last = nrm[:, -1:]
    o_ref[...] = n_last * lw_ref[...] + lb2_ref[...]
    pos = jax.lax.broadcasted_iota(jnp.int32, (SP, HD), 0).astype(jnp.float32)
    col = jax.lax.broadcasted_iota(jnp.int32, (SP, HD), 1)
    fidx = jnp.where(col >= HD // 2, col - HD // 2, col).astype(jnp.float32)
    inv = jnp.exp(fidx * (-2.0 * math.log(10000.0) / HD))
    ang = pos * inv
    ct_ref[...] = jnp.cos(ang)
    st_ref[...] = jnp.sin(ang)


def _rms_mm_body(x_ref, g_ref, w_ref, o_ref, xn_ref):
    @pl.when(pl.program_id(0) == 0)
    def _():
        xn_ref[...] = _rms(x_ref[...], g_ref[...])

    o_ref[...] = _dot_t(xn_ref[...], w_ref[...])


def _rot(x):
    return jnp.concatenate([-x[:, HD // 2:], x[:, :HD // 2]], axis=1)


def _attn_body(q_ref, k_ref, v_ref, ct_ref, st_ref, ow_ref, r_ref, o_ref):
    h = pl.program_id(1)
    cos = ct_ref[...]
    sin = st_ref[...]
    q = q_ref[0] * (1.0 / math.sqrt(HD))
    qr = q * cos + _rot(q) * sin
    k = k_ref[0]
    kr = k * cos + _rot(k) * sin
    s = _dot_t(qr, kr)
    row = jax.lax.broadcasted_iota(jnp.int32, (SP, SP), 0)
    col = jax.lax.broadcasted_iota(jnp.int32, (SP, SP), 1)
    s = s + jnp.where(col > row, NEG, 0.0)
    m = jnp.max(s, axis=1, keepdims=True)
    p = jnp.exp(s - m)
    a = p * (1.0 / jnp.sum(p, axis=1, keepdims=True))
    o = jax.lax.dot_general(a, v_ref[0], (((1,), (0,)), ((), ())),
                            preferred_element_type=jnp.float32)
    contrib = _dot_t(o, ow_ref[...])

    @pl.when(h == 0)
    def _():
        o_ref[0] = r_ref[0] + contrib

    @pl.when(h != 0)
    def _():
        o_ref[0] = o_ref[0] + contrib


def _ffn_body(x_ref, g_ref, wg_ref, wu_ref, wd_ref, o_ref, xn_ref):
    j = pl.program_id(0)

    @pl.when(j == 0)
    def _():
        xn_ref[...] = _rms(x_ref[...], g_ref[...])

    xn = xn_ref[...]
    g = _dot_t(xn, wg_ref[...])
    u = _dot_t(xn, wu_ref[...])
    a = g * jax.lax.logistic(g) * u
    contrib = _dot_t(a, wd_ref[...])

    @pl.when(j == 0)
    def _():
        o_ref[...] = x_ref[...] + contrib

    @pl.when(j != 0)
    def _():
        o_ref[...] = o_ref[...] + contrib


def _moe_body(x_ref, g_ref, rw_ref, sg_ref, su_ref, sd_ref,
              eg_ref, eu_ref, ed_ref, o_ref, xn_ref, gates_ref):
    j = pl.program_id(0)

    @pl.when(j == 0)
    def _():
        xn = _rms(x_ref[...], g_ref[...])
        xn_ref[...] = xn
        logits = _dot_t(xn, rw_ref[...])
        mx = jnp.max(logits, axis=1, keepdims=True)
        ex = jnp.exp(logits - mx)
        sm = ex * (1.0 / jnp.sum(ex, axis=1, keepdims=True))
        cols = jax.lax.broadcasted_iota(jnp.int32, (T, E), 1)
        i1 = jnp.argmax(sm, axis=1)
        oh1 = cols == i1[:, None]
        m1 = jnp.max(sm, axis=1, keepdims=True)
        sm2 = jnp.where(oh1, -jnp.inf, sm)
        i2 = jnp.argmax(sm2, axis=1)
        oh2 = cols == i2[:, None]
        m2 = jnp.max(sm2, axis=1, keepdims=True)
        gates_ref[...] = jnp.where(oh1, m1, 0.0) + jnp.where(oh2, m2, 0.0)
        o_ref[...] = x_ref[...]

    xn = xn_ref[...]

    @pl.when(j < NS)
    def _():
        g = _dot_t(xn, sg_ref[...])
        u = _dot_t(xn, su_ref[...])
        a = g * jax.lax.logistic(g) * u
        o_ref[...] = o_ref[...] + _dot_t(a, sd_ref[...])

    @pl.when(j >= NS)
    def _():
        e = j - NS
        gates = gates_ref[...]
        cols = jax.lax.broadcasted_iota(jnp.int32, (T, E), 1)
        we = jnp.sum(jnp.where(cols == e, gates, 0.0), axis=1, keepdims=True)
        g = _dot_t(xn, eg_ref[0])
        u = _dot_t(xn, eu_ref[0])
        a = g * jax.lax.logistic(g) * u
        ye = _dot_t(a, ed_ref[0])
        o_ref[...] = o_ref[...] + we * ye


def _head_body(x_ref, g_ref, w_ref, b_ref, o_ref):
    xn = _rms(x_ref[...], g_ref[...])
    o_ref[...] = _dot_t(xn, w_ref[...]) + b_ref[...]


def _full(shape):
    return pl.BlockSpec(shape, lambda *a: (0,) * len(shape))


def _embed(hist_pad, emb):
    return pl.pallas_call(
        _embed_body,
        out_shape=[jax.ShapeDtypeStruct((T, D), jnp.float32),
                   jax.ShapeDtypeStruct((SP, HD), jnp.float32),
                   jax.ShapeDtypeStruct((SP, HD), jnp.float32)],
        in_specs=[_full((T, 1))] + [_full(s) for s in
                  [(1, 64), (1, 64), (1, 64), (1, 64), (1, D), (1, D)]],
        out_specs=[_full((T, D)), _full((SP, HD)), _full((SP, HD))],
    )(hist_pad,
      emb['conv_w'].reshape(1, 64), emb['conv_b'].reshape(1, 64),
      emb['ln_g'].reshape(1, 64), emb['ln_b'].reshape(1, 64),
      emb['lin_w'].reshape(1, D), emb['lin_b'].reshape(1, D))


def _rms_mm(x, g, w, bn=512):
    n = w.shape[0]
    return pl.pallas_call(
        _rms_mm_body,
        grid=(n // bn,),
        out_shape=jax.ShapeDtypeStruct((T, n), jnp.float32),
        in_specs=[_full((T, D)), _full((1, D)),
                  pl.BlockSpec((bn, D), lambda j: (j, 0))],
        out_specs=pl.BlockSpec((T, bn), lambda j: (0, j)),
        scratch_shapes=[pltpu.VMEM((T, D), jnp.float32)],
        compiler_params=pltpu.CompilerParams(dimension_semantics=("arbitrary",)),
    )(x, g.reshape(1, D), w)


def _attention(qkv, ct, st, ow, r):
    spec_q = pl.BlockSpec((1, SP, HD), lambda b, h: (b, 0, h))
    spec_k = pl.BlockSpec((1, SP, HD), lambda b, h: (b, 0, H + h))
    spec_v = pl.BlockSpec((1, SP, HD), lambda b, h: (b, 0, 2 * H + h))
    return pl.pallas_call(
        _attn_body,
        grid=(B, H),
        out_shape=jax.ShapeDtypeStruct((B, SP, D), jnp.float32),
        in_specs=[spec_q, spec_k, spec_v,
                  _full((SP, HD)), _full((SP, HD)),
                  pl.BlockSpec((D, HD), lambda b, h: (0, h)),
                  pl.BlockSpec((1, SP, D), lambda b, h: (b, 0, 0))],
        out_specs=pl.BlockSpec((1, SP, D), lambda b, h: (b, 0, 0)),
        compiler_params=pltpu.CompilerParams(
            dimension_semantics=("arbitrary", "arbitrary")),
    )(qkv, qkv, qkv, ct, st, ow, r)


def _ffn(x, g, wg, wu, wd, bf=512):
    ff = wg.shape[0]
    return pl.pallas_call(
        _ffn_body,
        grid=(ff // bf,),
        out_shape=jax.ShapeDtypeStruct((T, D), jnp.float32),
        in_specs=[_full((T, D)), _full((1, D)),
                  pl.BlockSpec((bf, D), lambda j: (j, 0)),
                  pl.BlockSpec((bf, D), lambda j: (j, 0)),
                  pl.BlockSpec((D, bf), lambda j: (0, j))],
        out_specs=_full((T, D)),
        scratch_shapes=[pltpu.VMEM((T, D), jnp.float32)],
        compiler_params=pltpu.CompilerParams(dimension_semantics=("arbitrary",)),
    )(x, g.reshape(1, D), wg, wu, wd)


def _moe(x, g, rw, sg, su, sd, eg, eu, ed):
    mf = eg.shape[1]
    sb = sg.shape[0] // NS
    eix = lambda j: (jnp.maximum(j - NS, 0), 0, 0)
    six = lambda j: (jnp.minimum(j, NS - 1), 0)
    return pl.pallas_call(
        _moe_body,
        grid=(NS + E,),
        out_shape=jax.ShapeDtypeStruct((T, D), jnp.float32),
        in_specs=[_full((T, D)), _full((1, D)), _full((E, D)),
                  pl.BlockSpec((sb, D), six),
                  pl.BlockSpec((sb, D), six),
                  pl.BlockSpec((D, sb), lambda j: (0, jnp.minimum(j, NS - 1))),
                  pl.BlockSpec((1, mf, D), eix),
                  pl.BlockSpec((1, mf, D), eix),
                  pl.BlockSpec((1, D, mf), eix)],
        out_specs=_full((T, D)),
        scratch_shapes=[pltpu.VMEM((T, D), jnp.float32),
                        pltpu.VMEM((T, E), jnp.float32)],
        compiler_params=pltpu.CompilerParams(dimension_semantics=("arbitrary",)),
    )(x, g.reshape(1, D), rw, sg, su, sd, eg, eu, ed)


def _head(x, g, w, b):
    n = w.shape[0]
    return pl.pallas_call(
        _head_body,
        out_shape=jax.ShapeDtypeStruct((T, n), jnp.float32),
        in_specs=[_full((T, D)), _full((1, D)), _full((n, D)), _full((1, n))],
        out_specs=_full((T, n)),
    )(x, g.reshape(1, D), w, b.reshape(1, n))


def kernel(history_data, emb, layers, final_norm, od_w, od_b):
    hist_pad = jnp.pad(history_data, ((0, 0), (0, SP - S))).reshape(T, 1)
    h, ct, st = _embed(hist_pad, emb)

    for lp in layers:
        wqkv = jnp.concatenate([lp['q_w'], lp['k_w'], lp['v_w']], axis=0)
        qkv = _rms_mm(h, lp['attn_norm'], wqkv)
        h = _attention(qkv.reshape(B, SP, 3 * D), ct, st,
                       lp['o_w'], h.reshape(B, SP, D)).reshape(T, D)
        if 'router_w' in lp:
            h = _moe(h, lp['ffn_norm'], lp['router_w'],
                     lp['s_gate'], lp['s_up'], lp['s_down'],
                     lp['e_gate'], lp['e_up'], lp['e_down'])
        else:
            h = _ffn(h, lp['ffn_norm'], lp['gate_w'], lp['up_w'], lp['down_w'])

    y = _head(h, final_norm, od_w, od_b)
    return y.reshape(B, SP, S)[:, :S, :]
